# baseline (speedup 1.0000x reference)
import functools

import jax
import jax.numpy as jnp
from jax import lax
from jax.experimental import pallas as pl
from jax.experimental.pallas import tpu as pltpu
from jax.experimental.pallas import tpu_sc as plsc

GAMMA = 12.0
Q = 2016
N = 256
D = 64
S = 64
QB = 48
NUM_BLOCKS = Q // QB

SC_CORES = 2
SC_SUBCORES = 16
SC_WORKERS = SC_CORES * SC_SUBCORES
ROWS_PER_WORKER = N // SC_WORKERS


@functools.cache
def _make_sc_gather():
    @functools.partial(
        pl.kernel,
        out_type=jax.ShapeDtypeStruct((N, D), jnp.float32),
        mesh=plsc.VectorSubcoreMesh(core_axis_name="c", subcore_axis_name="s"),
        scratch_types=[
            pltpu.VMEM((ROWS_PER_WORKER,), jnp.int32),
            pltpu.VMEM((ROWS_PER_WORKER, D), jnp.float32),
            pltpu.SemaphoreType.DMA,
        ],
        compiler_params=pltpu.CompilerParams(use_tc_tiling_on_sc=False),
    )
    def _sc_gather(table_hbm, idx_hbm, out_hbm, idx_v, rows_v, sem):
        wid = lax.axis_index("s") * SC_CORES + lax.axis_index("c")
        base = wid * ROWS_PER_WORKER
        pltpu.sync_copy(idx_hbm.at[pl.ds(base, ROWS_PER_WORKER)], idx_v)
        pltpu.async_copy(table_hbm.at[idx_v], rows_v, sem).wait()
        pltpu.sync_copy(rows_v, out_hbm.at[pl.ds(base, ROWS_PER_WORKER)])

    return _sc_gather


def _tc_body(sub_ref, rel_ref, nbet_ref, st_ref, en_ref, inv_ref, out_ref):
    i = pl.program_id(0)
    obj = sub_ref[...] + rel_ref[...]
    acc = jnp.zeros((QB, N), jnp.float32)
    for d in range(D):
        acc = acc + jnp.abs(obj[:, d : d + 1] - nbet_ref[d : d + 1, :])
    x = GAMMA - acc

    rows = i * QB + lax.broadcasted_iota(jnp.int32, (S, QB), 1)
    mask = (rows >= st_ref[...]) & (rows < en_ref[...])
    w = jnp.where(mask, inv_ref[...], 0.0)
    contrib = jnp.dot(
        w, x, preferred_element_type=jnp.float32, precision=lax.Precision.HIGHEST
    )

    @pl.when(i == 0)
    def _():
        out_ref[...] = contrib

    @pl.when(i > 0)
    def _():
        out_ref[...] += contrib


def kernel(sub_emb, rel_emb, target, node_emb, graph_batch_x, num_neigh):
    del target
    idx = graph_batch_x.astype(jnp.int32)
    node_batch = jnp.take(node_emb, idx, axis=0)
    nbet = node_batch.T

    cnt = num_neigh.astype(jnp.int32)
    ends = jnp.cumsum(cnt)
    starts = ends - cnt
    inv = 1.0 / jnp.maximum(cnt.astype(jnp.float32), 1e-12)
    starts2 = starts.reshape(S, 1)
    ends2 = ends.reshape(S, 1)
    inv2 = inv.reshape(S, 1)

    return pl.pallas_call(
        _tc_body,
        grid=(NUM_BLOCKS,),
        in_specs=[
            pl.BlockSpec((QB, D), lambda i: (i, 0)),
            pl.BlockSpec((QB, D), lambda i: (i, 0)),
            pl.BlockSpec((D, N), lambda i: (0, 0)),
            pl.BlockSpec((S, 1), lambda i: (0, 0)),
            pl.BlockSpec((S, 1), lambda i: (0, 0)),
            pl.BlockSpec((S, 1), lambda i: (0, 0)),
        ],
        out_specs=pl.BlockSpec((S, N), lambda i: (0, 0)),
        out_shape=jax.ShapeDtypeStruct((S, N), jnp.float32),
    )(sub_emb, rel_emb, nbet, starts2, ends2, inv2)


# speedup vs baseline: 1.1284x; 1.1284x over previous
import functools

import jax
import jax.numpy as jnp
from jax import lax
from jax.experimental import pallas as pl
from jax.experimental.pallas import tpu as pltpu
from jax.experimental.pallas import tpu_sc as plsc

GAMMA = 12.0
Q = 2016
N = 256
D = 64
S = 64
QB = 96
NUM_BLOCKS = Q // QB

SC_CORES = 2
SC_SUBCORES = 16
SC_WORKERS = SC_CORES * SC_SUBCORES
ROWS_PER_WORKER = N // SC_WORKERS


@functools.cache
def _make_sc_gather():
    @functools.partial(
        pl.kernel,
        out_type=jax.ShapeDtypeStruct((N, D), jnp.float32),
        mesh=plsc.VectorSubcoreMesh(core_axis_name="c", subcore_axis_name="s"),
        scratch_types=[
            pltpu.VMEM((16,), jnp.int32),
            pltpu.VMEM((ROWS_PER_WORKER, D), jnp.float32),
            pltpu.SemaphoreType.DMA,
        ],
    )
    def _sc_gather(table_hbm, idx_hbm, out_hbm, idx_v, rows_v, sem):
        wid = lax.axis_index("s") * SC_CORES + lax.axis_index("c")
        base = wid * ROWS_PER_WORKER
        pltpu.sync_copy(
            idx_hbm.at[pl.ds(base, ROWS_PER_WORKER)],
            idx_v.at[pl.ds(0, ROWS_PER_WORKER)],
        )
        vec = idx_v[...]
        copies = [
            pltpu.async_copy(
                table_hbm.at[pl.ds(vec[j], 1), :],
                rows_v.at[pl.ds(j, 1), :],
                sem,
            )
            for j in range(ROWS_PER_WORKER)
        ]
        for c in copies:
            c.wait()
        pltpu.sync_copy(rows_v, out_hbm.at[pl.ds(base, ROWS_PER_WORKER)])

    return _sc_gather


def _tc_body(sub_ref, rel_ref, nbet_ref, st_ref, en_ref, inv_ref, out_ref):
    i = pl.program_id(0)
    obj = sub_ref[...] + rel_ref[...]

    acc = jnp.zeros((QB, N), jnp.float32)
    for d in range(D):
        acc = acc + jnp.abs(obj[:, d : d + 1] - nbet_ref[d : d + 1, :])
    x = GAMMA - acc

    rows = i * QB + lax.broadcasted_iota(jnp.int32, (S, QB), 1)
    mask = (rows >= st_ref[...]) & (rows < en_ref[...])
    w = jnp.where(mask, inv_ref[...], 0.0)
    contrib = jnp.dot(
        w, x, preferred_element_type=jnp.float32, precision=lax.Precision.HIGHEST
    )

    @pl.when(i == 0)
    def _():
        out_ref[...] = contrib

    @pl.when(i > 0)
    def _():
        out_ref[...] += contrib


def kernel(sub_emb, rel_emb, target, node_emb, graph_batch_x, num_neigh):
    del target
    idx = graph_batch_x.astype(jnp.int32)
    node_batch = _make_sc_gather()(node_emb, idx)
    nbet = node_batch.T

    cnt = num_neigh.astype(jnp.int32)
    ends = jnp.cumsum(cnt)
    starts = ends - cnt
    inv = 1.0 / jnp.maximum(cnt.astype(jnp.float32), 1e-12)
    starts2 = starts.reshape(S, 1)
    ends2 = ends.reshape(S, 1)
    inv2 = inv.reshape(S, 1)

    return pl.pallas_call(
        _tc_body,
        grid=(NUM_BLOCKS,),
        in_specs=[
            pl.BlockSpec((QB, D), lambda i: (i, 0)),
            pl.BlockSpec((QB, D), lambda i: (i, 0)),
            pl.BlockSpec((D, N), lambda i: (0, 0)),
            pl.BlockSpec((S, 1), lambda i: (0, 0)),
            pl.BlockSpec((S, 1), lambda i: (0, 0)),
            pl.BlockSpec((S, 1), lambda i: (0, 0)),
        ],
        out_specs=pl.BlockSpec((S, N), lambda i: (0, 0)),
        out_shape=jax.ShapeDtypeStruct((S, N), jnp.float32),
    )(sub_emb, rel_emb, nbet, starts2, ends2, inv2)


# speedup vs baseline: 1.1299x; 1.0013x over previous
import functools

import jax
import jax.numpy as jnp
from jax import lax
from jax.experimental import pallas as pl
from jax.experimental.pallas import tpu as pltpu
from jax.experimental.pallas import tpu_sc as plsc

GAMMA = 12.0
Q = 2016
N = 256
D = 64
S = 64
QB = 96
NUM_BLOCKS = Q // QB

SC_CORES = 2
SC_SUBCORES = 16
SC_WORKERS = SC_CORES * SC_SUBCORES
ROWS_PER_WORKER = N // SC_WORKERS


@functools.cache
def _make_sc_gather():
    @functools.partial(
        pl.kernel,
        out_type=jax.ShapeDtypeStruct((N, D), jnp.float32),
        mesh=plsc.VectorSubcoreMesh(core_axis_name="c", subcore_axis_name="s"),
        scratch_types=[
            pltpu.VMEM((16,), jnp.int32),
            pltpu.VMEM((ROWS_PER_WORKER, D), jnp.float32),
            pltpu.SemaphoreType.DMA,
        ],
        compiler_params=pltpu.CompilerParams(needs_layout_passes=True),
    )
    def _sc_gather(table_hbm, idx_hbm, out_hbm, idx_v, rows_v, sem):
        wid = lax.axis_index("s") * SC_CORES + lax.axis_index("c")
        base = wid * ROWS_PER_WORKER
        pltpu.sync_copy(
            idx_hbm.at[pl.ds(base, ROWS_PER_WORKER)],
            idx_v.at[pl.ds(0, ROWS_PER_WORKER)],
        )
        vec = idx_v[...]
        copies = [
            pltpu.async_copy(
                table_hbm.at[pl.ds(vec[j], 1), :],
                rows_v.at[pl.ds(j, 1), :],
                sem,
            )
            for j in range(ROWS_PER_WORKER)
        ]
        for c in copies:
            c.wait()
        pltpu.sync_copy(rows_v, out_hbm.at[pl.ds(base, ROWS_PER_WORKER)])

    return _sc_gather


def _tc_body(sub_ref, rel_ref, nbet_ref, st_ref, en_ref, inv_ref, out_ref):
    i = pl.program_id(0)
    obj = sub_ref[...] + rel_ref[...]

    acc = jnp.zeros((QB, N), jnp.float32)
    for d in range(D):
        acc = acc + jnp.abs(obj[:, d : d + 1] - nbet_ref[d : d + 1, :])
    x = GAMMA - acc

    rows = i * QB + lax.broadcasted_iota(jnp.int32, (S, QB), 1)
    mask = (rows >= st_ref[...]) & (rows < en_ref[...])
    w = jnp.where(mask, inv_ref[...], 0.0)
    contrib = jnp.dot(
        w, x, preferred_element_type=jnp.float32, precision=lax.Precision.HIGHEST
    )

    @pl.when(i == 0)
    def _():
        out_ref[...] = contrib

    @pl.when(i > 0)
    def _():
        out_ref[...] += contrib


def kernel(sub_emb, rel_emb, target, node_emb, graph_batch_x, num_neigh):
    del target
    idx = graph_batch_x.astype(jnp.int32)
    node_batch = _make_sc_gather()(node_emb, idx)
    nbet = node_batch.T

    cnt = num_neigh.astype(jnp.int32)
    ends = jnp.cumsum(cnt)
    starts = ends - cnt
    inv = 1.0 / jnp.maximum(cnt.astype(jnp.float32), 1e-12)
    starts2 = starts.reshape(S, 1)
    ends2 = ends.reshape(S, 1)
    inv2 = inv.reshape(S, 1)

    return pl.pallas_call(
        _tc_body,
        grid=(NUM_BLOCKS,),
        in_specs=[
            pl.BlockSpec((QB, D), lambda i: (i, 0)),
            pl.BlockSpec((QB, D), lambda i: (i, 0)),
            pl.BlockSpec((D, N), lambda i: (0, 0)),
            pl.BlockSpec((S, 1), lambda i: (0, 0)),
            pl.BlockSpec((S, 1), lambda i: (0, 0)),
            pl.BlockSpec((S, 1), lambda i: (0, 0)),
        ],
        out_specs=pl.BlockSpec((S, N), lambda i: (0, 0)),
        out_shape=jax.ShapeDtypeStruct((S, N), jnp.float32),
    )(sub_emb, rel_emb, nbet, starts2, ends2, inv2)


# speedup vs baseline: 1.4486x; 1.2820x over previous
import functools

import jax
import jax.numpy as jnp
from jax import lax
from jax.experimental import pallas as pl
from jax.experimental.pallas import tpu as pltpu
from jax.experimental.pallas import tpu_sc as plsc

GAMMA = 12.0
Q = 2016
N = 256
D = 64
S = 64
QB = 96
NUM_BLOCKS = Q // QB

SC_CORES = 2
SC_SUBCORES = 16
SC_WORKERS = SC_CORES * SC_SUBCORES
ROWS_PER_WORKER = N // SC_WORKERS


@functools.cache
def _make_sc_gather():
    @functools.partial(
        pl.kernel,
        out_type=jax.ShapeDtypeStruct((SC_WORKERS, ROWS_PER_WORKER * D), jnp.float32),
        mesh=plsc.VectorSubcoreMesh(core_axis_name="c", subcore_axis_name="s"),
        scratch_types=[
            pltpu.VMEM((16,), jnp.int32),
            [pltpu.VMEM((D, 128), jnp.float32) for _ in range(ROWS_PER_WORKER)],
            pltpu.VMEM((ROWS_PER_WORKER * D,), jnp.float32),
            pltpu.SemaphoreType.DMA,
        ],
        compiler_params=pltpu.CompilerParams(needs_layout_passes=False),
    )
    def _sc_gather(table_t_hbm, idx_hbm, out_hbm, idx_v, tiles, rows_v, sem):
        wid = lax.axis_index("s") * SC_CORES + lax.axis_index("c")
        base = wid * ROWS_PER_WORKER
        pltpu.sync_copy(
            idx_hbm.at[pl.ds(base, ROWS_PER_WORKER)],
            idx_v.at[pl.ds(0, ROWS_PER_WORKER)],
        )
        vec = idx_v[...]
        tile_ids = vec // 128
        lanes = vec - tile_ids * 128
        copies = [
            pltpu.async_copy(
                table_t_hbm.at[:, pl.ds(pl.multiple_of(tile_ids[j] * 128, 128), 128)],
                tiles[j],
                sem,
            )
            for j in range(ROWS_PER_WORKER)
        ]
        for c in copies:
            c.wait()
        for j in range(ROWS_PER_WORKER):
            col_idx = jnp.full((16,), lanes[j], jnp.int32)
            for c in range(D // 16):
                row_idx = lax.iota(jnp.int32, 16) + c * 16
                vals = plsc.load_gather(tiles[j], [row_idx, col_idx])
                rows_v[pl.ds(j * D + c * 16, 16)] = vals
        pltpu.sync_copy(rows_v, out_hbm.at[wid])

    return _sc_gather


def _tc_body(sub_ref, rel_ref, nbet_ref, st_ref, en_ref, inv_ref, out_ref):
    i = pl.program_id(0)
    obj = sub_ref[...] + rel_ref[...]

    acc = jnp.zeros((QB, N), jnp.float32)
    for d in range(D):
        acc = acc + jnp.abs(obj[:, d : d + 1] - nbet_ref[d : d + 1, :])
    x = GAMMA - acc

    rows = i * QB + lax.broadcasted_iota(jnp.int32, (S, QB), 1)
    mask = (rows >= st_ref[...]) & (rows < en_ref[...])
    w = jnp.where(mask, inv_ref[...], 0.0)
    contrib = jnp.dot(
        w, x, preferred_element_type=jnp.float32, precision=lax.Precision.HIGHEST
    )

    @pl.when(i == 0)
    def _():
        out_ref[...] = contrib

    @pl.when(i > 0)
    def _():
        out_ref[...] += contrib


def kernel(sub_emb, rel_emb, target, node_emb, graph_batch_x, num_neigh):
    del target
    idx = graph_batch_x.astype(jnp.int32)
    gathered = _make_sc_gather()(node_emb.T, idx)
    nbet = gathered.reshape(N, D).T

    cnt = num_neigh.astype(jnp.int32)
    ends = jnp.cumsum(cnt)
    starts = ends - cnt
    inv = 1.0 / jnp.maximum(cnt.astype(jnp.float32), 1e-12)
    starts2 = starts.reshape(S, 1)
    ends2 = ends.reshape(S, 1)
    inv2 = inv.reshape(S, 1)

    return pl.pallas_call(
        _tc_body,
        grid=(NUM_BLOCKS,),
        in_specs=[
            pl.BlockSpec((QB, D), lambda i: (i, 0)),
            pl.BlockSpec((QB, D), lambda i: (i, 0)),
            pl.BlockSpec((D, N), lambda i: (0, 0)),
            pl.BlockSpec((S, 1), lambda i: (0, 0)),
            pl.BlockSpec((S, 1), lambda i: (0, 0)),
            pl.BlockSpec((S, 1), lambda i: (0, 0)),
        ],
        out_specs=pl.BlockSpec((S, N), lambda i: (0, 0)),
        out_shape=jax.ShapeDtypeStruct((S, N), jnp.float32),
    )(sub_emb, rel_emb, nbet, starts2, ends2, inv2)


# speedup vs baseline: 1.5990x; 1.1038x over previous
import functools

import jax
import jax.numpy as jnp
from jax import lax
from jax.experimental import pallas as pl
from jax.experimental.pallas import tpu as pltpu
from jax.experimental.pallas import tpu_sc as plsc

GAMMA = 12.0
Q = 2016
N = 256
D = 64
S = 64
QB = 336
NUM_BLOCKS = Q // QB

SC_CORES = 2
SC_SUBCORES = 16
SC_WORKERS = SC_CORES * SC_SUBCORES
ROWS_PER_WORKER = N // SC_WORKERS


@functools.cache
def _make_sc_gather():
    @functools.partial(
        pl.kernel,
        out_type=jax.ShapeDtypeStruct((SC_WORKERS, ROWS_PER_WORKER * D), jnp.float32),
        mesh=plsc.VectorSubcoreMesh(core_axis_name="c", subcore_axis_name="s"),
        scratch_types=[
            pltpu.VMEM((16,), jnp.int32),
            [pltpu.VMEM((D, 128), jnp.float32) for _ in range(ROWS_PER_WORKER)],
            pltpu.VMEM((ROWS_PER_WORKER * D,), jnp.float32),
            pltpu.SemaphoreType.DMA,
        ],
        compiler_params=pltpu.CompilerParams(needs_layout_passes=False),
    )
    def _sc_gather(table_t_hbm, idx_hbm, out_hbm, idx_v, tiles, rows_v, sem):
        wid = lax.axis_index("s") * SC_CORES + lax.axis_index("c")
        base = wid * ROWS_PER_WORKER
        pltpu.sync_copy(
            idx_hbm.at[pl.ds(base, ROWS_PER_WORKER)],
            idx_v.at[pl.ds(0, ROWS_PER_WORKER)],
        )
        vec = idx_v[...]
        tile_ids = vec // 128
        lanes = vec - tile_ids * 128
        copies = [
            pltpu.async_copy(
                table_t_hbm.at[:, pl.ds(pl.multiple_of(tile_ids[j] * 128, 128), 128)],
                tiles[j],
                sem,
            )
            for j in range(ROWS_PER_WORKER)
        ]
        for c in copies:
            c.wait()
        for j in range(ROWS_PER_WORKER):
            col_idx = jnp.full((16,), lanes[j], jnp.int32)
            for c in range(D // 16):
                row_idx = lax.iota(jnp.int32, 16) + c * 16
                vals = plsc.load_gather(tiles[j], [row_idx, col_idx])
                rows_v[pl.ds(j * D + c * 16, 16)] = vals
        pltpu.sync_copy(rows_v, out_hbm.at[wid])

    return _sc_gather


def _tc_body(sub_ref, rel_ref, nbet_ref, st_ref, en_ref, inv_ref, out_ref):
    i = pl.program_id(0)
    obj = sub_ref[...] + rel_ref[...]

    acc = jnp.zeros((QB, N), jnp.float32)
    for d in range(D):
        acc = acc + jnp.abs(obj[:, d : d + 1] - nbet_ref[d : d + 1, :])
    x = GAMMA - acc

    rows = i * QB + lax.broadcasted_iota(jnp.int32, (S, QB), 1)
    mask = (rows >= st_ref[...]) & (rows < en_ref[...])
    w = jnp.where(mask, inv_ref[...], 0.0)
    contrib = jnp.dot(
        w, x, preferred_element_type=jnp.float32, precision=lax.Precision.HIGHEST
    )

    @pl.when(i == 0)
    def _():
        out_ref[...] = contrib

    @pl.when(i > 0)
    def _():
        out_ref[...] += contrib


def kernel(sub_emb, rel_emb, target, node_emb, graph_batch_x, num_neigh):
    del target
    idx = graph_batch_x.astype(jnp.int32)
    gathered = _make_sc_gather()(node_emb.T, idx)
    nbet = gathered.reshape(N, D).T

    cnt = num_neigh.astype(jnp.int32)
    ends = jnp.cumsum(cnt)
    starts = ends - cnt
    inv = 1.0 / jnp.maximum(cnt.astype(jnp.float32), 1e-12)
    starts2 = starts.reshape(S, 1)
    ends2 = ends.reshape(S, 1)
    inv2 = inv.reshape(S, 1)

    return pl.pallas_call(
        _tc_body,
        grid=(NUM_BLOCKS,),
        in_specs=[
            pl.BlockSpec((QB, D), lambda i: (i, 0)),
            pl.BlockSpec((QB, D), lambda i: (i, 0)),
            pl.BlockSpec((D, N), lambda i: (0, 0)),
            pl.BlockSpec((S, 1), lambda i: (0, 0)),
            pl.BlockSpec((S, 1), lambda i: (0, 0)),
            pl.BlockSpec((S, 1), lambda i: (0, 0)),
        ],
        out_specs=pl.BlockSpec((S, N), lambda i: (0, 0)),
        out_shape=jax.ShapeDtypeStruct((S, N), jnp.float32),
    )(sub_emb, rel_emb, nbet, starts2, ends2, inv2)
